# Initial kernel scaffold; baseline (speedup 1.0000x reference)
import jax
import jax.numpy as jnp
from jax.experimental import pallas as pl


def kernel(physical_features, physical_edge_index, physical_edge_attr, virtual_features, virtual_edge_index, virtual_edge_attr, virtual_node_idx, params):
    raise NotImplementedError("write your pallas kernel here")


# speedup vs baseline: 19.8934x; 19.8934x over previous
import functools

import jax
import jax.numpy as jnp
from jax import lax
from jax.experimental import pallas as pl
from jax.experimental.pallas import tpu as pltpu
from jax.experimental.pallas import tpu_sc as plsc

N = 10000
NPAD = 10240
HID = 128
HEADS = 4
B = 128
ROWS_PT = NPAD // 16
R = 2048
GRID = NPAD // R


def _mx_block(s, d):
    ms = jnp.max(s, axis=0)
    md = jnp.max(d, axis=0)
    return jnp.concatenate(
        [jnp.broadcast_to(ms[:, None], (4, 128)),
         jnp.broadcast_to(md[:, None], (4, 128))], axis=0)


def _pre_tail(h, a_ref, b_ref, h_ref, as_ref, ad_ref, mx_ref):
    h_ref[...] = h
    s = jnp.dot(h, a_ref[...], preferred_element_type=jnp.float32)
    d = jnp.dot(h, b_ref[...], preferred_element_type=jnp.float32)
    as_ref[...] = s
    ad_ref[...] = d
    cur = _mx_block(s, d)
    i = pl.program_id(0)

    @pl.when(i == 0)
    def _():
        mx_ref[...] = cur

    @pl.when(i > 0)
    def _():
        mx_ref[...] = jnp.maximum(mx_ref[...], cur)


def _enc_pre_body(x_ref, w_ref, a_ref, b_ref, h_ref, as_ref, ad_ref, mx_ref):
    h = jnp.dot(x_ref[...], w_ref[...], preferred_element_type=jnp.float32)
    _pre_tail(h, a_ref, b_ref, h_ref, as_ref, ad_ref, mx_ref)


def _combine(raw_ref, den_ref, bias_ref):
    raw = raw_ref[...]
    den = den_ref[...]
    acc = raw[0] / (den[:, 0:1] + 1e-16)
    for hd in range(1, 4):
        acc = acc + raw[hd] / (den[:, hd:hd + 1] + 1e-16)
    return jnp.maximum(acc * 0.25 + bias_ref[...], 0.0)


def _enc_mid_body(raw_ref, den_ref, bias_ref, w_ref, a_ref, b_ref,
                  h_ref, as_ref, ad_ref, mx_ref):
    y = _combine(raw_ref, den_ref, bias_ref)
    h = jnp.dot(y, w_ref[...], preferred_element_type=jnp.float32)
    _pre_tail(h, a_ref, b_ref, h_ref, as_ref, ad_ref, mx_ref)


def _enc_post_body(raw_ref, den_ref, bias_ref, pw_ref, pb_ref, out_ref):
    y = _combine(raw_ref, den_ref, bias_ref)
    out_ref[...] = (jnp.dot(y, pw_ref[...], preferred_element_type=jnp.float32)
                    + pb_ref[...])


def _rb(i):
    return (i, 0)


_F32 = jnp.float32


def _enc_pre(x, w, a, b):
    return pl.pallas_call(
        _enc_pre_body,
        grid=(GRID,),
        in_specs=[
            pl.BlockSpec((R, HID), _rb),
            pl.BlockSpec((HID, 512), lambda i: (0, 0)),
            pl.BlockSpec((512, 4), lambda i: (0, 0)),
            pl.BlockSpec((512, 4), lambda i: (0, 0)),
        ],
        out_specs=[
            pl.BlockSpec((R, 512), _rb),
            pl.BlockSpec((R, 4), _rb),
            pl.BlockSpec((R, 4), _rb),
            pl.BlockSpec((8, 128), lambda i: (0, 0)),
        ],
        out_shape=[
            jax.ShapeDtypeStruct((NPAD, 512), _F32),
            jax.ShapeDtypeStruct((NPAD, 4), _F32),
            jax.ShapeDtypeStruct((NPAD, 4), _F32),
            jax.ShapeDtypeStruct((8, 128), _F32),
        ],
    )(x, w, a, b)


def _enc_mid(raw, den, bias, w, a, b):
    return pl.pallas_call(
        _enc_mid_body,
        grid=(GRID,),
        in_specs=[
            pl.BlockSpec((4, R, 128), lambda i: (0, i, 0)),
            pl.BlockSpec((R, 4), _rb),
            pl.BlockSpec((1, 128), lambda i: (0, 0)),
            pl.BlockSpec((HID, 512), lambda i: (0, 0)),
            pl.BlockSpec((512, 4), lambda i: (0, 0)),
            pl.BlockSpec((512, 4), lambda i: (0, 0)),
        ],
        out_specs=[
            pl.BlockSpec((R, 512), _rb),
            pl.BlockSpec((R, 4), _rb),
            pl.BlockSpec((R, 4), _rb),
            pl.BlockSpec((8, 128), lambda i: (0, 0)),
        ],
        out_shape=[
            jax.ShapeDtypeStruct((NPAD, 512), _F32),
            jax.ShapeDtypeStruct((NPAD, 4), _F32),
            jax.ShapeDtypeStruct((NPAD, 4), _F32),
            jax.ShapeDtypeStruct((8, 128), _F32),
        ],
    )(raw, den, bias, w, a, b)


def _enc_post(raw, den, bias, pw, pb):
    return pl.pallas_call(
        _enc_post_body,
        grid=(GRID,),
        in_specs=[
            pl.BlockSpec((4, R, 128), lambda i: (0, i, 0)),
            pl.BlockSpec((R, 4), _rb),
            pl.BlockSpec((1, 128), lambda i: (0, 0)),
            pl.BlockSpec((HID, HID), lambda i: (0, 0)),
            pl.BlockSpec((1, 128), lambda i: (0, 0)),
        ],
        out_specs=[pl.BlockSpec((R, HID), _rb)],
        out_shape=[jax.ShapeDtypeStruct((NPAD, HID), _F32)],
    )(raw, den, bias, pw, pb)[0]


def _make_edge_kernel(e_pad):
    ct = e_pad // (16 * B)
    mesh = plsc.VectorSubcoreMesh(core_axis_name="c", subcore_axis_name="s",
                                  num_cores=2, num_subcores=16)

    @functools.partial(
        pl.kernel,
        out_type=(jax.ShapeDtypeStruct((4, NPAD, 128), _F32),
                  jax.ShapeDtypeStruct((4, NPAD), _F32)),
        mesh=mesh,
        scratch_types=[
            pltpu.VMEM((B,), jnp.int32),
            pltpu.VMEM((B,), jnp.int32),
            pltpu.VMEM((B,), jnp.int32),
            pltpu.VMEM((B,), jnp.int32),
            pltpu.VMEM((B,), _F32),
            pltpu.VMEM((B,), _F32),
            pltpu.VMEM((B,), _F32),
            pltpu.VMEM((B, 128), _F32),
            pltpu.VMEM((128,), _F32),
            pltpu.VMEM((128,), _F32),
            pltpu.VMEM((64, 128), _F32),
            pltpu.VMEM((ROWS_PT,), _F32),
            pltpu.VMEM_SHARED((NPAD, 128), _F32),
            pltpu.VMEM_SHARED((NPAD,), _F32),
            pltpu.SemaphoreType.DMA,
        ],
        compiler_params=pltpu.CompilerParams(needs_layout_passes=False),
    )
    def edge_kernel(h4, as4, ad4, mx, src_e, dst_e, raw_out, den_out,
                    src_v, dst_v, row_v, dv_v, e_v, asb, adb, hbuf, cs_v, cd_v,
                    zb, zd, shared_out, shared_den, sem):
        c = lax.axis_index("c")
        s = lax.axis_index("s")
        ebase = s * (e_pad // 16)
        rbase = s * ROWS_PT
        zero16 = jnp.zeros((16,), _F32)

        for r in range(64):
            for j in range(8):
                zb[r, pl.ds(16 * j, 16)] = zero16

        @pl.loop(0, ROWS_PT // 16)
        def _(r):
            zd[pl.ds(r * 16, 16)] = zero16

        @pl.loop(0, 2)
        def _(p):
            head = c * 2 + p
            pltpu.sync_copy(mx.at[head], cs_v)
            pltpu.sync_copy(mx.at[head + 4], cd_v)
            t = cs_v[pl.ds(0, 16)] + cd_v[pl.ds(0, 16)]
            cvec = jnp.maximum(t, 0.2 * t)

            @pl.loop(0, ROWS_PT // 64)
            def _(kk):
                pltpu.sync_copy(zb, shared_out.at[pl.ds(rbase + kk * 64, 64)])

            pltpu.sync_copy(zd, shared_den.at[pl.ds(rbase, ROWS_PT)])
            plsc.subcore_barrier()

            @pl.loop(0, ct)
            def _(ci):
                base = ebase + ci * B
                pltpu.sync_copy(src_e.at[pl.ds(base, B)], src_v)
                pltpu.sync_copy(dst_e.at[pl.ds(base, B)], dst_v)
                for g in range(B // 16):
                    sl = pl.ds(16 * g, 16)
                    row_v[sl] = src_v[sl] * 4 + head
                    dv_v[sl] = dst_v[sl] * 4 + head
                c1 = pltpu.async_copy(as4.at[row_v], asb, sem)
                c2 = pltpu.async_copy(ad4.at[dv_v], adb, sem)
                c3 = pltpu.async_copy(h4.at[row_v], hbuf, sem)
                c1.wait()
                c2.wait()
                c3.wait()
                for g in range(B // 16):
                    sl = pl.ds(16 * g, 16)
                    av = asb[sl] + adb[sl]
                    av = jnp.maximum(av, 0.2 * av)
                    e_v[sl] = jnp.exp(av - cvec)

                @pl.loop(0, B, unroll=8)
                def _(r):
                    rr = jnp.full((16,), 0, jnp.int32) + r
                    eb = plsc.load_gather(e_v, [rr])
                    for j in range(8):
                        sl = pl.ds(16 * j, 16)
                        hbuf[r, sl] = hbuf[r, sl] * eb

                pltpu.sync_copy(hbuf, shared_out.at[dst_v], add=True)
                pltpu.sync_copy(e_v, shared_den.at[dst_v], add=True)

            plsc.subcore_barrier()
            pltpu.sync_copy(shared_out.at[pl.ds(rbase, ROWS_PT)],
                            raw_out.at[head, pl.ds(rbase, ROWS_PT)])
            pltpu.sync_copy(shared_den.at[pl.ds(rbase, ROWS_PT)],
                            den_out.at[head, pl.ds(rbase, ROWS_PT)])
            plsc.subcore_barrier()

    return edge_kernel


def _tail_body(pe_ref, cur_ref, wq, bq, wk, bk, wv, bv, wo, bo,
               m0w, m0b, m1w, m1b, m2w, m2b, c0w, c0b, c1w, c1b, c2w, c2b,
               map_ref, bw_ref):
    pe = pe_ref[...]
    cur = cur_ref[...]
    q = jnp.dot(cur, wq[...], preferred_element_type=_F32) + bq[...]
    kk = jnp.dot(pe, wk[...], preferred_element_type=_F32) + bk[...]
    vv = jnp.dot(pe, wv[...], preferred_element_type=_F32) + bv[...]
    hsel = (lax.broadcasted_iota(jnp.int32, (128, 8), 0) // 16
            == lax.broadcasted_iota(jnp.int32, (128, 8), 1))
    qmask = jnp.where(hsel, jnp.broadcast_to(q.reshape(128, 1), (128, 8)), 0.0)
    sc = jnp.dot(kk, qmask, preferred_element_type=_F32) * 0.25
    rid = lax.broadcasted_iota(jnp.int32, (NPAD, 8), 0)
    sc = jnp.where(rid < N, sc, -1e30)
    m = jnp.max(sc, axis=0, keepdims=True)
    ex = jnp.exp(sc - m)
    den = jnp.sum(ex, axis=0, keepdims=True)
    w = ex / den
    w_exp = jnp.dot(w, jnp.where(hsel, 1.0, 0.0).T,
                    preferred_element_type=_F32)
    attn = jnp.sum(vv * w_exp, axis=0).reshape(1, 128)
    attn = jnp.dot(attn, wo[...], preferred_element_type=_F32) + bo[...]
    fused = jnp.concatenate([cur, attn], axis=1)
    h0 = jnp.maximum(jnp.dot(fused, m0w[...], preferred_element_type=_F32)
                     + m0b[...], 0.0)
    h1 = jnp.maximum(jnp.dot(h0, m1w[...], preferred_element_type=_F32)
                     + m1b[...], 0.0)
    map_ref[...] = jnp.dot(h1, m2w[...], preferred_element_type=_F32) + m2b[...]
    g0 = jnp.maximum(jnp.dot(fused, c0w[...], preferred_element_type=_F32)
                     + c0b[...], 0.0)
    g1 = jnp.maximum(jnp.dot(g0, c1w[...], preferred_element_type=_F32)
                     + c1b[...], 0.0)
    bw_ref[...] = jnp.dot(g1, c2w[...], preferred_element_type=_F32) + c2b[...]


def _tail(pe, cur, attn_p, map_p, bw_p):
    args = [pe, cur]
    for nm in ("q", "k", "v", "o"):
        args.append(attn_p[nm]["W"])
        args.append(attn_p[nm]["b"].reshape(1, -1))
    for lp in map_p:
        args.append(lp["W"])
        args.append(lp["b"].reshape(1, -1))
    for lp in bw_p:
        args.append(lp["W"])
        args.append(lp["b"].reshape(1, -1))
    return pl.pallas_call(
        _tail_body,
        out_shape=[jax.ShapeDtypeStruct((1, N), _F32),
                   jax.ShapeDtypeStruct((1, 10), _F32)],
    )(*args)


def _att_mat(att):
    return (jnp.eye(4, dtype=_F32)[:, None, :] * att[:, :, None]).reshape(512, 4)


def _prep_edges(ei):
    e = ei.shape[1]
    loop = jnp.arange(N, dtype=jnp.int32)
    src = jnp.concatenate([ei[0].astype(jnp.int32), loop])
    dst = jnp.concatenate([ei[1].astype(jnp.int32), loop])
    tot = e + N
    e_pad = -(-tot // 2048) * 2048
    fill = N + (jnp.arange(e_pad - tot, dtype=jnp.int32) % 16)
    src = jnp.concatenate([src, fill])
    dst = jnp.concatenate([dst, fill])
    return src, dst, e_pad


def _pad_rows(x):
    return jnp.pad(x, ((0, NPAD - N), (0, 0)))


def _encode(x, src, dst, edge_k, p):
    lp = p["layers"]
    h, a_s, a_d, mx = _enc_pre(x, lp[0]["W"], _att_mat(lp[0]["att_src"]),
                               _att_mat(lp[0]["att_dst"]))
    raw, den = edge_k(h.reshape(NPAD * 4, 128), a_s.reshape(NPAD * 4),
                      a_d.reshape(NPAD * 4), mx, src, dst)
    for li in (1, 2):
        h, a_s, a_d, mx = _enc_mid(raw, den.T,
                                   lp[li - 1]["bias"].reshape(1, 128),
                                   lp[li]["W"], _att_mat(lp[li]["att_src"]),
                                   _att_mat(lp[li]["att_dst"]))
        raw, den = edge_k(h.reshape(NPAD * 4, 128), a_s.reshape(NPAD * 4),
                          a_d.reshape(NPAD * 4), mx, src, dst)
    return _enc_post(raw, den.T, lp[2]["bias"].reshape(1, 128),
                     p["proj"]["W"], p["proj"]["b"].reshape(1, 128))


def kernel(physical_features, physical_edge_index, physical_edge_attr,
           virtual_features, virtual_edge_index, virtual_edge_attr,
           virtual_node_idx, params):
    src_p, dst_p, ep_p = _prep_edges(physical_edge_index)
    src_v, dst_v, ep_v = _prep_edges(virtual_edge_index)
    ek_p = _make_edge_kernel(ep_p)
    ek_v = _make_edge_kernel(ep_v)
    pe = _encode(_pad_rows(physical_features), src_p, dst_p, ek_p,
                 params["phys"])
    ve = _encode(_pad_rows(virtual_features), src_v, dst_v, ek_v,
                 params["virt"])
    cur = lax.dynamic_slice_in_dim(ve, virtual_node_idx, 1, axis=0)
    map_l, bw_l = _tail(pe, cur, params["attn"], params["map_head"],
                        params["bw_head"])
    return (map_l[0], bw_l[0])


# speedup vs baseline: 23.1003x; 1.1612x over previous
import functools

import jax
import jax.numpy as jnp
from jax import lax
from jax.experimental import pallas as pl
from jax.experimental.pallas import tpu as pltpu
from jax.experimental.pallas import tpu_sc as plsc

N = 10000
NPAD = 10240
HID = 128
HEADS = 4
B = 128
ROWS_PT = NPAD // 16
R = 2048
GRID = NPAD // R


def _mx_block(s, d):
    ms = jnp.max(s, axis=0)
    md = jnp.max(d, axis=0)
    return jnp.concatenate(
        [jnp.broadcast_to(ms[:, None], (4, 128)),
         jnp.broadcast_to(md[:, None], (4, 128))], axis=0)


def _pre_tail(h, a_ref, b_ref, h_ref, as_ref, ad_ref, mx_ref):
    h_ref[...] = h
    s = jnp.dot(h, a_ref[...], preferred_element_type=jnp.float32)
    d = jnp.dot(h, b_ref[...], preferred_element_type=jnp.float32)
    as_ref[...] = s
    ad_ref[...] = d
    cur = _mx_block(s, d)
    i = pl.program_id(0)

    @pl.when(i == 0)
    def _():
        mx_ref[...] = cur

    @pl.when(i > 0)
    def _():
        mx_ref[...] = jnp.maximum(mx_ref[...], cur)


def _enc_pre_body(x_ref, w_ref, a_ref, b_ref, h_ref, as_ref, ad_ref, mx_ref):
    h = jnp.dot(x_ref[...], w_ref[...], preferred_element_type=jnp.float32)
    _pre_tail(h, a_ref, b_ref, h_ref, as_ref, ad_ref, mx_ref)


def _combine(raw_ref, den_ref, bias_ref):
    raw = raw_ref[...]
    den = den_ref[...]
    acc = raw[0] / (den[:, 0:1] + 1e-16)
    for hd in range(1, 4):
        acc = acc + raw[hd] / (den[:, hd:hd + 1] + 1e-16)
    return jnp.maximum(acc * 0.25 + bias_ref[...], 0.0)


def _enc_mid_body(raw_ref, den_ref, bias_ref, w_ref, a_ref, b_ref,
                  h_ref, as_ref, ad_ref, mx_ref):
    y = _combine(raw_ref, den_ref, bias_ref)
    h = jnp.dot(y, w_ref[...], preferred_element_type=jnp.float32)
    _pre_tail(h, a_ref, b_ref, h_ref, as_ref, ad_ref, mx_ref)


def _enc_post_body(raw_ref, den_ref, bias_ref, pw_ref, pb_ref, out_ref):
    y = _combine(raw_ref, den_ref, bias_ref)
    out_ref[...] = (jnp.dot(y, pw_ref[...], preferred_element_type=jnp.float32)
                    + pb_ref[...])


def _rb(i):
    return (i, 0)


_F32 = jnp.float32


def _enc_pre(x, w, a, b):
    return pl.pallas_call(
        _enc_pre_body,
        grid=(GRID,),
        in_specs=[
            pl.BlockSpec((R, HID), _rb),
            pl.BlockSpec((HID, 512), lambda i: (0, 0)),
            pl.BlockSpec((512, 4), lambda i: (0, 0)),
            pl.BlockSpec((512, 4), lambda i: (0, 0)),
        ],
        out_specs=[
            pl.BlockSpec((R, 512), _rb),
            pl.BlockSpec((R, 4), _rb),
            pl.BlockSpec((R, 4), _rb),
            pl.BlockSpec((8, 128), lambda i: (0, 0)),
        ],
        out_shape=[
            jax.ShapeDtypeStruct((NPAD, 512), _F32),
            jax.ShapeDtypeStruct((NPAD, 4), _F32),
            jax.ShapeDtypeStruct((NPAD, 4), _F32),
            jax.ShapeDtypeStruct((8, 128), _F32),
        ],
    )(x, w, a, b)


def _enc_mid(raw, den, bias, w, a, b):
    return pl.pallas_call(
        _enc_mid_body,
        grid=(GRID,),
        in_specs=[
            pl.BlockSpec((4, R, 128), lambda i: (0, i, 0)),
            pl.BlockSpec((R, 4), _rb),
            pl.BlockSpec((1, 128), lambda i: (0, 0)),
            pl.BlockSpec((HID, 512), lambda i: (0, 0)),
            pl.BlockSpec((512, 4), lambda i: (0, 0)),
            pl.BlockSpec((512, 4), lambda i: (0, 0)),
        ],
        out_specs=[
            pl.BlockSpec((R, 512), _rb),
            pl.BlockSpec((R, 4), _rb),
            pl.BlockSpec((R, 4), _rb),
            pl.BlockSpec((8, 128), lambda i: (0, 0)),
        ],
        out_shape=[
            jax.ShapeDtypeStruct((NPAD, 512), _F32),
            jax.ShapeDtypeStruct((NPAD, 4), _F32),
            jax.ShapeDtypeStruct((NPAD, 4), _F32),
            jax.ShapeDtypeStruct((8, 128), _F32),
        ],
    )(raw, den, bias, w, a, b)


def _enc_post(raw, den, bias, pw, pb):
    return pl.pallas_call(
        _enc_post_body,
        grid=(GRID,),
        in_specs=[
            pl.BlockSpec((4, R, 128), lambda i: (0, i, 0)),
            pl.BlockSpec((R, 4), _rb),
            pl.BlockSpec((1, 128), lambda i: (0, 0)),
            pl.BlockSpec((HID, HID), lambda i: (0, 0)),
            pl.BlockSpec((1, 128), lambda i: (0, 0)),
        ],
        out_specs=[pl.BlockSpec((R, HID), _rb)],
        out_shape=[jax.ShapeDtypeStruct((NPAD, HID), _F32)],
    )(raw, den, bias, pw, pb)[0]


def _make_edge_kernel(e_pad):
    ct = e_pad // (16 * B)
    mesh = plsc.VectorSubcoreMesh(core_axis_name="c", subcore_axis_name="s",
                                  num_cores=2, num_subcores=16)

    @functools.partial(
        pl.kernel,
        out_type=(jax.ShapeDtypeStruct((4, NPAD, 128), _F32),
                  jax.ShapeDtypeStruct((4, NPAD), _F32)),
        mesh=mesh,
        scratch_types=[
            pltpu.VMEM((B,), jnp.int32),
            pltpu.VMEM((B,), jnp.int32),
            pltpu.VMEM((B,), jnp.int32),
            pltpu.VMEM((B,), jnp.int32),
            pltpu.VMEM((B,), jnp.int32),
            pltpu.VMEM((B,), jnp.int32),
            pltpu.VMEM((B,), _F32),
            pltpu.VMEM((B,), _F32),
            pltpu.VMEM((B,), _F32),
            pltpu.VMEM((B,), _F32),
            pltpu.VMEM((B, 128), _F32),
            pltpu.VMEM((B, 128), _F32),
            pltpu.VMEM((128,), _F32),
            pltpu.VMEM((128,), _F32),
            pltpu.VMEM((64, 128), _F32),
            pltpu.VMEM((ROWS_PT,), _F32),
            pltpu.VMEM_SHARED((NPAD, 128), _F32),
            pltpu.VMEM_SHARED((NPAD,), _F32),
            pltpu.SemaphoreType.DMA,
            pltpu.SemaphoreType.DMA,
        ],
        compiler_params=pltpu.CompilerParams(needs_layout_passes=False),
    )
    def edge_kernel(h4, as4, ad4, mx, src_e, dst_e, raw_out, den_out,
                    src_v, dst_v0, dst_v1, row_v0, row_v1, dv_v,
                    e_v0, e_v1, asb, adb, hbuf0, hbuf1, cs_v, cd_v,
                    zb, zd, shared_out, shared_den, sem0, sem1):
        c = lax.axis_index("c")
        s = lax.axis_index("s")
        ebase = s * (e_pad // 16)
        rbase = s * ROWS_PT
        zero16 = jnp.zeros((16,), _F32)

        for r in range(64):
            for j in range(8):
                zb[r, pl.ds(16 * j, 16)] = zero16

        @pl.loop(0, ROWS_PT // 16)
        def _(r):
            zd[pl.ds(r * 16, 16)] = zero16

        slots = ((dst_v0, row_v0, e_v0, hbuf0, sem0),
                 (dst_v1, row_v1, e_v1, hbuf1, sem1))

        @pl.loop(0, 2)
        def _(p):
            head = c * 2 + p
            pltpu.sync_copy(mx.at[head], cs_v)
            pltpu.sync_copy(mx.at[head + 4], cd_v)
            t = cs_v[pl.ds(0, 16)] + cd_v[pl.ds(0, 16)]
            cvec = jnp.maximum(t, 0.2 * t)

            @pl.loop(0, ROWS_PT // 64)
            def _(kk):
                pltpu.sync_copy(zb, shared_out.at[pl.ds(rbase + kk * 64, 64)])

            pltpu.sync_copy(zd, shared_den.at[pl.ds(rbase, ROWS_PT)])
            plsc.subcore_barrier()

            def prefetch(ci, slot):
                dst_vx, row_vx, e_vx, hbufx, semx = slot
                base = ebase + ci * B
                pltpu.sync_copy(src_e.at[pl.ds(base, B)], src_v)
                pltpu.sync_copy(dst_e.at[pl.ds(base, B)], dst_vx)
                for g in range(B // 16):
                    sl = pl.ds(16 * g, 16)
                    row_vx[sl] = src_v[sl] * 4 + head
                    dv_v[sl] = dst_vx[sl] * 4 + head
                c1 = pltpu.async_copy(as4.at[row_vx], asb, semx)
                c2 = pltpu.async_copy(ad4.at[dv_v], adb, semx)
                c1.wait()
                c2.wait()
                for g in range(B // 16):
                    sl = pl.ds(16 * g, 16)
                    av = asb[sl] + adb[sl]
                    av = jnp.maximum(av, 0.2 * av)
                    e_vx[sl] = jnp.exp(av - cvec)
                pltpu.async_copy(h4.at[row_vx], hbufx, semx)

            def process(slot):
                dst_vx, row_vx, e_vx, hbufx, semx = slot
                pltpu.make_async_copy(h4.at[row_vx], hbufx, semx).wait()

                @pl.loop(0, B, unroll=8)
                def _(r):
                    rr = jnp.full((16,), 0, jnp.int32) + r
                    eb = plsc.load_gather(e_vx, [rr])
                    for j in range(8):
                        sl = pl.ds(16 * j, 16)
                        hbufx[r, sl] = hbufx[r, sl] * eb

                pltpu.sync_copy(hbufx, shared_out.at[dst_vx], add=True)
                pltpu.sync_copy(e_vx, shared_den.at[dst_vx], add=True)

            prefetch(0, slots[0])

            @pl.loop(0, ct // 2)
            def _(ko):
                for b in range(2):
                    ci = ko * 2 + b
                    nci = ci + 1

                    @pl.when(nci < ct)
                    def _():
                        prefetch(nci, slots[1 - b])

                    process(slots[b])

            plsc.subcore_barrier()
            pltpu.sync_copy(shared_out.at[pl.ds(rbase, ROWS_PT)],
                            raw_out.at[head, pl.ds(rbase, ROWS_PT)])
            pltpu.sync_copy(shared_den.at[pl.ds(rbase, ROWS_PT)],
                            den_out.at[head, pl.ds(rbase, ROWS_PT)])
            plsc.subcore_barrier()

    return edge_kernel


def _tail_body(pe_ref, cur_ref, wq, bq, wk, bk, wv, bv, wo, bo,
               m0w, m0b, m1w, m1b, m2w, m2b, c0w, c0b, c1w, c1b, c2w, c2b,
               map_ref, bw_ref):
    pe = pe_ref[...]
    cur = cur_ref[...]
    q = jnp.dot(cur, wq[...], preferred_element_type=_F32) + bq[...]
    kk = jnp.dot(pe, wk[...], preferred_element_type=_F32) + bk[...]
    vv = jnp.dot(pe, wv[...], preferred_element_type=_F32) + bv[...]
    hsel = (lax.broadcasted_iota(jnp.int32, (128, 8), 0) // 16
            == lax.broadcasted_iota(jnp.int32, (128, 8), 1))
    qmask = jnp.where(hsel, jnp.broadcast_to(q.reshape(128, 1), (128, 8)), 0.0)
    sc = jnp.dot(kk, qmask, preferred_element_type=_F32) * 0.25
    rid = lax.broadcasted_iota(jnp.int32, (NPAD, 8), 0)
    sc = jnp.where(rid < N, sc, -1e30)
    m = jnp.max(sc, axis=0, keepdims=True)
    ex = jnp.exp(sc - m)
    den = jnp.sum(ex, axis=0, keepdims=True)
    w = ex / den
    w_exp = jnp.dot(w, jnp.where(hsel, 1.0, 0.0).T,
                    preferred_element_type=_F32)
    attn = jnp.sum(vv * w_exp, axis=0).reshape(1, 128)
    attn = jnp.dot(attn, wo[...], preferred_element_type=_F32) + bo[...]
    fused = jnp.concatenate([cur, attn], axis=1)
    h0 = jnp.maximum(jnp.dot(fused, m0w[...], preferred_element_type=_F32)
                     + m0b[...], 0.0)
    h1 = jnp.maximum(jnp.dot(h0, m1w[...], preferred_element_type=_F32)
                     + m1b[...], 0.0)
    map_ref[...] = jnp.dot(h1, m2w[...], preferred_element_type=_F32) + m2b[...]
    g0 = jnp.maximum(jnp.dot(fused, c0w[...], preferred_element_type=_F32)
                     + c0b[...], 0.0)
    g1 = jnp.maximum(jnp.dot(g0, c1w[...], preferred_element_type=_F32)
                     + c1b[...], 0.0)
    bw_ref[...] = jnp.dot(g1, c2w[...], preferred_element_type=_F32) + c2b[...]


def _tail(pe, cur, attn_p, map_p, bw_p):
    args = [pe, cur]
    for nm in ("q", "k", "v", "o"):
        args.append(attn_p[nm]["W"])
        args.append(attn_p[nm]["b"].reshape(1, -1))
    for lp in map_p:
        args.append(lp["W"])
        args.append(lp["b"].reshape(1, -1))
    for lp in bw_p:
        args.append(lp["W"])
        args.append(lp["b"].reshape(1, -1))
    return pl.pallas_call(
        _tail_body,
        out_shape=[jax.ShapeDtypeStruct((1, N), _F32),
                   jax.ShapeDtypeStruct((1, 10), _F32)],
    )(*args)


def _att_mat(att):
    return (jnp.eye(4, dtype=_F32)[:, None, :] * att[:, :, None]).reshape(512, 4)


def _prep_edges(ei):
    e = ei.shape[1]
    loop = jnp.arange(N, dtype=jnp.int32)
    src = jnp.concatenate([ei[0].astype(jnp.int32), loop])
    dst = jnp.concatenate([ei[1].astype(jnp.int32), loop])
    tot = e + N
    e_pad = -(-tot // 4096) * 4096
    fill = N + (jnp.arange(e_pad - tot, dtype=jnp.int32) % 16)
    src = jnp.concatenate([src, fill])
    dst = jnp.concatenate([dst, fill])
    return src, dst, e_pad


def _pad_rows(x):
    return jnp.pad(x, ((0, NPAD - N), (0, 0)))


def _encode(x, src, dst, edge_k, p):
    lp = p["layers"]
    h, a_s, a_d, mx = _enc_pre(x, lp[0]["W"], _att_mat(lp[0]["att_src"]),
                               _att_mat(lp[0]["att_dst"]))
    raw, den = edge_k(h.reshape(NPAD * 4, 128), a_s.reshape(NPAD * 4),
                      a_d.reshape(NPAD * 4), mx, src, dst)
    for li in (1, 2):
        h, a_s, a_d, mx = _enc_mid(raw, den.T,
                                   lp[li - 1]["bias"].reshape(1, 128),
                                   lp[li]["W"], _att_mat(lp[li]["att_src"]),
                                   _att_mat(lp[li]["att_dst"]))
        raw, den = edge_k(h.reshape(NPAD * 4, 128), a_s.reshape(NPAD * 4),
                          a_d.reshape(NPAD * 4), mx, src, dst)
    return _enc_post(raw, den.T, lp[2]["bias"].reshape(1, 128),
                     p["proj"]["W"], p["proj"]["b"].reshape(1, 128))


def kernel(physical_features, physical_edge_index, physical_edge_attr,
           virtual_features, virtual_edge_index, virtual_edge_attr,
           virtual_node_idx, params):
    src_p, dst_p, ep_p = _prep_edges(physical_edge_index)
    src_v, dst_v, ep_v = _prep_edges(virtual_edge_index)
    ek_p = _make_edge_kernel(ep_p)
    ek_v = _make_edge_kernel(ep_v)
    pe = _encode(_pad_rows(physical_features), src_p, dst_p, ek_p,
                 params["phys"])
    ve = _encode(_pad_rows(virtual_features), src_v, dst_v, ek_v,
                 params["virt"])
    cur = lax.dynamic_slice_in_dim(ve, virtual_node_idx, 1, axis=0)
    map_l, bw_l = _tail(pe, cur, params["attn"], params["map_head"],
                        params["bw_head"])
    return (map_l[0], bw_l[0])
